# Initial kernel scaffold; baseline (speedup 1.0000x reference)
import jax
import jax.numpy as jnp
from jax.experimental import pallas as pl


def kernel(u, x, in_proj_w, in_proj_b, out_proj_w, out_proj_b, conv1_w, ln_w, ln_b, conv2_w, conv2_b):
    raise NotImplementedError("write your pallas kernel here")


# speedup vs baseline: 8741.5508x; 8741.5508x over previous
import jax
import jax.numpy as jnp
from jax.experimental import pallas as pl


def _corres_kernel(ln_b_ref, c2w_ref, c2b_ref, out_ref):
    ln_b = ln_b_ref[0, :]
    c2w = c2w_ref[0, 0]
    c2b = c2b_ref[0, 0]
    g = ln_b * 0.5 * (1.0 + jax.lax.erf(ln_b / jnp.sqrt(jnp.float32(2.0))))
    h = c2w * g + c2b
    out_ref[:, :] = jnp.broadcast_to(jax.nn.sigmoid(h)[None, :], out_ref.shape)


def kernel(u, x, in_proj_w, in_proj_b, out_proj_w, out_proj_b,
           conv1_w, ln_w, ln_b, conv2_w, conv2_b):
    B, N, C = u.shape
    out = pl.pallas_call(
        _corres_kernel,
        out_shape=jax.ShapeDtypeStruct((B, N), jnp.float32),
    )(ln_b, conv2_w, conv2_b.reshape(1, 1))
    return out
